# baseline (speedup 1.0000x reference)
import functools

import numpy as np
import jax
import jax.numpy as jnp
from jax import lax
from jax.experimental import pallas as pl
from jax.experimental.pallas import tpu as pltpu
from jax.experimental.pallas import tpu_sc as plsc

_NUM_SAMPLE = 4096

_NUM_CORES = 2
_NUM_SUBCORES = 16
_NUM_WORKERS = _NUM_CORES * _NUM_SUBCORES
_LANE = 128
_CHUNK_ROWS = 4


@functools.lru_cache(maxsize=None)
def _row_sample_indices(B: int, N: int) -> np.ndarray:
    cpu = jax.local_devices(backend="cpu")[0]
    with jax.ensure_compile_time_eval(), jax.default_device(cpu):
        keys = jax.random.split(jax.random.key(42), B)
        idx = jax.vmap(lambda k: jax.random.permutation(k, N)[:_NUM_SAMPLE])(keys)
    idx = np.asarray(jax.device_get(idx)).astype(np.int64)
    rows = idx + (np.arange(B, dtype=np.int64) * N)[:, None]
    return rows.reshape(-1, _LANE).astype(np.int32)


@functools.lru_cache(maxsize=None)
def _build_gather(R: int, C: int):
    per_w = R // _NUM_WORKERS
    rows_per_w = per_w // _LANE
    assert per_w * _NUM_WORKERS == R and rows_per_w * _LANE == per_w
    n_chunks = rows_per_w // _CHUNK_ROWS
    assert n_chunks * _CHUNK_ROWS == rows_per_w
    mesh = plsc.VectorSubcoreMesh(core_axis_name="c", subcore_axis_name="s")

    @functools.partial(
        pl.kernel,
        out_type=jax.ShapeDtypeStruct((R, C), jnp.float32),
        mesh=mesh,
        scratch_types=[
            pltpu.SMEM((_CHUNK_ROWS, _LANE), jnp.int32),
            pltpu.VMEM_SHARED(
                (_NUM_SUBCORES, _CHUNK_ROWS, _LANE), jnp.int32
            ),
            pltpu.SemaphoreType.DMA,
        ],
    )
    def gather_kernel(table_hbm, idx_hbm, out_hbm, idx_s, idx_sp, sem):
        sid = lax.axis_index("s")
        wid = sid * _NUM_CORES + lax.axis_index("c")
        base = wid * per_w
        idx_row0 = wid * rows_per_w

        def chunk_body(k, carry):
            pltpu.sync_copy(
                idx_hbm.at[pl.ds(idx_row0 + k * _CHUNK_ROWS, _CHUNK_ROWS)],
                idx_sp.at[sid],
            )
            pltpu.sync_copy(idx_sp.at[sid], idx_s)
            off = base + k * (_CHUNK_ROWS * _LANE)

            def row_body(a, c2):
                def issue(b, c3):
                    r = idx_s[a, b]
                    pltpu.async_copy(
                        table_hbm.at[pl.ds(r, 1)],
                        out_hbm.at[pl.ds(off + a * _LANE + b, 1)],
                        sem,
                    )
                    return c3

                return lax.fori_loop(0, _LANE, issue, c2)

            return lax.fori_loop(0, _CHUNK_ROWS, row_body, carry)

        lax.fori_loop(0, n_chunks, chunk_body, 0)
        pltpu.make_async_copy(
            table_hbm.at[pl.ds(0, per_w)], out_hbm.at[pl.ds(base, per_w)], sem
        ).wait()

    return gather_kernel


def kernel(points):
    B, N, C = points.shape
    row_idx = jnp.asarray(_row_sample_indices(B, N))
    table = points.reshape(B * N, C)
    out = _build_gather(B * _NUM_SAMPLE, C)(table, row_idx)
    return out.reshape(B, _NUM_SAMPLE, C)


# speedup vs baseline: 1.2947x; 1.2947x over previous
import functools

import numpy as np
import jax
import jax.numpy as jnp
from jax import lax
from jax.experimental import pallas as pl
from jax.experimental.pallas import tpu as pltpu
from jax.experimental.pallas import tpu_sc as plsc

_NUM_SAMPLE = 4096

_NUM_CORES = 2
_NUM_SUBCORES = 16
_NUM_WORKERS = _NUM_CORES * _NUM_SUBCORES
_LANE = 128
_CHUNK = 128


@functools.lru_cache(maxsize=None)
def _plan(B: int, N: int, C: int):
    cpu = jax.local_devices(backend="cpu")[0]
    with jax.ensure_compile_time_eval(), jax.default_device(cpu):
        keys = jax.random.split(jax.random.key(42), B)
        idx = jax.vmap(lambda k: jax.random.permutation(k, N)[:_NUM_SAMPLE])(keys)
    idx = np.asarray(jax.device_get(idx)).astype(np.int64)
    p = (idx + (np.arange(B, dtype=np.int64) * N)[:, None]).reshape(-1)
    f0 = p * C
    r0 = f0 // _LANE
    c0 = f0 % _LANE
    r1 = (f0 + C - 1) // _LANE

    R = p.shape[0]
    n_chunks = R // _CHUNK
    rowpairs = np.stack(
        [r0.reshape(n_chunks, _CHUNK), r1.reshape(n_chunks, _CHUNK)], axis=1
    ).reshape(-1)

    j = np.arange(C, dtype=np.int64)
    within = np.arange(R, dtype=np.int64) % _CHUNK
    half = (c0[:, None] + j[None, :]) // _LANE
    mrow = half * _CHUNK + within[:, None]
    mcol = (c0[:, None] + j[None, :]) % _LANE
    return (
        rowpairs.astype(np.int32),
        mrow.reshape(-1).astype(np.int32),
        mcol.reshape(-1).astype(np.int32),
    )


@functools.lru_cache(maxsize=None)
def _build_gather(R: int, C: int):
    per_w = R // _NUM_WORKERS
    n_chunks = per_w // _CHUNK
    assert per_w * _NUM_WORKERS == R and n_chunks * _CHUNK == per_w
    ew = per_w * C
    groups = _CHUNK * C // 16
    mesh = plsc.VectorSubcoreMesh(core_axis_name="c", subcore_axis_name="s")

    @functools.partial(
        pl.kernel,
        out_type=jax.ShapeDtypeStruct((R * C,), jnp.float32),
        mesh=mesh,
        scratch_types=[
            pltpu.VMEM((n_chunks * 2 * _CHUNK,), jnp.int32),
            pltpu.VMEM((ew,), jnp.int32),
            pltpu.VMEM((ew,), jnp.int32),
            pltpu.VMEM((2, 2 * _CHUNK, _LANE), jnp.float32),
            pltpu.VMEM((ew,), jnp.float32),
            pltpu.SemaphoreType.DMA,
            pltpu.SemaphoreType.DMA,
        ],
        compiler_params=pltpu.CompilerParams(needs_layout_passes=False),
    )
    def gather_kernel(
        table_hbm, rp_hbm, mrow_hbm, mcol_hbm, out_hbm,
        rp_v, mrow_v, mcol_v, rows_v, out_v, sem0, sem1,
    ):
        wid = lax.axis_index("s") * _NUM_CORES + lax.axis_index("c")
        pltpu.sync_copy(
            rp_hbm.at[pl.ds(wid * (n_chunks * 2 * _CHUNK), n_chunks * 2 * _CHUNK)],
            rp_v,
        )
        pltpu.sync_copy(mrow_hbm.at[pl.ds(wid * ew, ew)], mrow_v)
        pltpu.sync_copy(mcol_hbm.at[pl.ds(wid * ew, ew)], mcol_v)

        def fire(k, buf, sem):
            roff = k * 2 * _CHUNK
            pltpu.async_copy(
                table_hbm.at[rp_v.at[pl.ds(roff, _CHUNK)]],
                rows_v.at[buf, pl.ds(0, _CHUNK)],
                sem,
            )
            pltpu.async_copy(
                table_hbm.at[rp_v.at[pl.ds(roff + _CHUNK, _CHUNK)]],
                rows_v.at[buf, pl.ds(_CHUNK, _CHUNK)],
                sem,
            )

        def wait(buf, sem):
            pltpu.make_async_copy(
                table_hbm.at[pl.ds(0, 2 * _CHUNK)],
                rows_v.at[buf],
                sem,
            ).wait()

        fire(0, 0, sem0)

        for k in range(n_chunks):
            buf = k % 2
            if k + 1 < n_chunks:
                fire(k + 1, (k + 1) % 2, sem1 if (k + 1) % 2 else sem0)
            wait(buf, sem0 if buf == 0 else sem1)
            qbase = k * _CHUNK * C

            def extract(g, carry):
                q = qbase + g * 16
                bvec = jnp.zeros((16,), jnp.int32) + buf
                rvec = mrow_v[pl.ds(q, 16)]
                cvec = mcol_v[pl.ds(q, 16)]
                vals = plsc.load_gather(rows_v, [bvec, rvec, cvec])
                out_v[pl.ds(q, 16)] = vals
                return carry

            lax.fori_loop(0, groups, extract, 0)

        pltpu.sync_copy(out_v, out_hbm.at[pl.ds(wid * ew, ew)])

    return gather_kernel


def kernel(points):
    B, N, C = points.shape
    rowpairs, mrow, mcol = _plan(B, N, C)
    R = B * _NUM_SAMPLE
    table = points.reshape(B * N * C // _LANE, _LANE)
    out = _build_gather(R, C)(
        table, jnp.asarray(rowpairs), jnp.asarray(mrow), jnp.asarray(mcol)
    )
    return out.reshape(B, _NUM_SAMPLE, C)


# speedup vs baseline: 1.5077x; 1.1645x over previous
import functools

import numpy as np
import jax
import jax.numpy as jnp
from jax import lax
from jax.experimental import pallas as pl
from jax.experimental.pallas import tpu as pltpu
from jax.experimental.pallas import tpu_sc as plsc

_NUM_SAMPLE = 4096

_NUM_CORES = 2
_NUM_SUBCORES = 16
_NUM_WORKERS = _NUM_CORES * _NUM_SUBCORES
_CHUNK = 128


@functools.lru_cache(maxsize=None)
def _flat_sample_indices(B: int, N: int, C: int) -> np.ndarray:
    cpu = jax.local_devices(backend="cpu")[0]
    with jax.ensure_compile_time_eval(), jax.default_device(cpu):
        keys = jax.random.split(jax.random.key(42), B)
        idx = jax.vmap(lambda k: jax.random.permutation(k, N)[:_NUM_SAMPLE])(keys)
    idx = np.asarray(jax.device_get(idx)).astype(np.int64)
    rows = idx + (np.arange(B, dtype=np.int64) * N)[:, None]
    elems = rows.reshape(-1, 1) * C + np.arange(C, dtype=np.int64)
    return elems.reshape(-1).astype(np.int32)


@functools.lru_cache(maxsize=None)
def _build_flatten(B: int, N: int, C: int):
    w_per_b = _NUM_WORKERS // B
    assert w_per_b * B == _NUM_WORKERS and N % w_per_b == 0
    rows_w = N // w_per_b
    elems_w = rows_w * C
    ROWS = 200
    MACROS = ROWS * C // 48
    WRITE_EVERY = 25
    n_chunks = rows_w // ROWS
    span = WRITE_EVERY * ROWS * C
    assert n_chunks * ROWS == rows_w and n_chunks % WRITE_EVERY == 0
    assert MACROS * 48 == ROWS * C and span % 8 == 0
    mesh = plsc.VectorSubcoreMesh(core_axis_name="c", subcore_axis_name="s")

    @functools.partial(
        pl.kernel,
        out_type=jax.ShapeDtypeStruct((B * N * C,), jnp.float32),
        mesh=mesh,
        scratch_types=[
            pltpu.VMEM((2, ROWS, C), jnp.float32),
            pltpu.VMEM((span,), jnp.float32),
            pltpu.SemaphoreType.DMA,
            pltpu.SemaphoreType.DMA,
        ],
        compiler_params=pltpu.CompilerParams(needs_layout_passes=False),
    )
    def flatten_kernel(pts_hbm, out_hbm, vbuf, out_v, sem0, sem1):
        wid = lax.axis_index("s") * _NUM_CORES + lax.axis_index("c")
        b = wid // w_per_b
        h = wid % w_per_b
        row0 = h * rows_w
        obase = wid * elems_w

        iota = lax.iota(jnp.int32, 16)
        phases = []
        for p in range(3):
            q = iota + 16 * p
            iv = q // C
            phases.append((iv, q - iv * C))

        def fire(k, buf, sem):
            pltpu.async_copy(
                pts_hbm.at[b, pl.ds(row0 + k * ROWS, ROWS), :],
                vbuf.at[buf],
                sem,
            )

        def wait(buf, sem):
            pltpu.make_async_copy(
                pts_hbm.at[b, pl.ds(row0, ROWS), :], vbuf.at[buf], sem
            ).wait()

        fire(0, 0, sem0)

        def chunk_body(k, carry):
            par = lax.rem(k, 2)
            nxt_ok = k + 1 < n_chunks
            nxt_par = lax.rem(k + 1, 2)

            @pl.when(jnp.logical_and(nxt_ok, nxt_par == 0))
            def _():
                fire(k + 1, 0, sem0)

            @pl.when(jnp.logical_and(nxt_ok, nxt_par == 1))
            def _():
                fire(k + 1, 1, sem1)

            @pl.when(par == 0)
            def _():
                wait(0, sem0)

            @pl.when(par == 1)
            def _():
                wait(1, sem1)

            bvec = jnp.zeros((16,), jnp.int32) + par
            vbase = lax.rem(k, WRITE_EVERY) * (ROWS * C)

            def macro(m, c2):
                for p, (iv, jv) in enumerate(phases):
                    vals = plsc.load_gather(vbuf, [bvec, iv + 8 * m, jv])
                    out_v[pl.ds(vbase + m * 48 + p * 16, 16)] = vals
                return c2

            lax.fori_loop(0, MACROS, macro, 0)

            @pl.when(lax.rem(k, WRITE_EVERY) == WRITE_EVERY - 1)
            def _():
                pltpu.sync_copy(
                    out_v,
                    out_hbm.at[pl.ds(obase + (k // WRITE_EVERY) * span, span)],
                )

            return carry

        lax.fori_loop(0, n_chunks, chunk_body, 0)

    return flatten_kernel


@functools.lru_cache(maxsize=None)
def _build_gather(E: int):
    assert E % (_NUM_WORKERS * _CHUNK) == 0
    per_w = E // _NUM_WORKERS
    n_chunks = per_w // _CHUNK
    mesh = plsc.VectorSubcoreMesh(core_axis_name="c", subcore_axis_name="s")

    @functools.partial(
        pl.kernel,
        out_type=jax.ShapeDtypeStruct((E,), jnp.float32),
        mesh=mesh,
        scratch_types=[
            pltpu.VMEM((per_w,), jnp.int32),
            pltpu.VMEM((per_w,), jnp.float32),
            pltpu.SemaphoreType.DMA,
        ],
    )
    def gather_kernel(table_hbm, idx_hbm, out_hbm, idx_v, vals_v, sem):
        wid = lax.axis_index("s") * _NUM_CORES + lax.axis_index("c")
        base = wid * per_w
        pltpu.sync_copy(idx_hbm.at[pl.ds(base, per_w)], idx_v)

        def issue(c, carry):
            off = c * _CHUNK
            pltpu.async_copy(
                table_hbm.at[idx_v.at[pl.ds(off, _CHUNK)]],
                vals_v.at[pl.ds(off, _CHUNK)],
                sem,
            )
            return carry

        lax.fori_loop(0, n_chunks, issue, 0)
        pltpu.make_async_copy(table_hbm.at[pl.ds(0, per_w)], vals_v, sem).wait()
        pltpu.sync_copy(vals_v, out_hbm.at[pl.ds(base, per_w)])

    return gather_kernel


def kernel(points):
    B, N, C = points.shape
    flat_idx = jnp.asarray(_flat_sample_indices(B, N, C))
    table = _build_flatten(B, N, C)(points)
    out = _build_gather(B * _NUM_SAMPLE * C)(table, flat_idx)
    return out.reshape(B, _NUM_SAMPLE, C)
